# Initial kernel scaffold; baseline (speedup 1.0000x reference)
import jax
import jax.numpy as jnp
from jax.experimental import pallas as pl


def kernel(x, edge_index, self_feat, W1, b1, W2, b2, Wf1, bf1, Wf2, bf2):
    raise NotImplementedError("write your pallas kernel here")


# speedup vs baseline: 7.8187x; 7.8187x over previous
import functools

import jax
import jax.numpy as jnp
from jax import lax
from jax.experimental import pallas as pl
from jax.experimental.pallas import tpu as pltpu
from jax.experimental.pallas import tpu_sc as plsc

N = 10000
E = 320000
D1 = 112
D2 = 32
NC = 2
NS = 16
NW = NC * NS
CHUNK = 128
NCHUNKS = E // CHUNK
BASE_TRIPS = NCHUNKS // NW
EXTRA = NCHUNKS - BASE_TRIPS * NW
NP = 10112
ROWS_PER_TILE = NP // NS
BN = 1000


def _make_sc_scatter(d):
    mesh = plsc.VectorSubcoreMesh(core_axis_name="c", subcore_axis_name="s")

    @functools.partial(
        pl.kernel,
        mesh=mesh,
        out_type=jax.ShapeDtypeStruct((NC, NP, d), jnp.float32),
        scratch_types=[
            pltpu.VMEM((CHUNK,), jnp.int32),
            pltpu.VMEM((CHUNK,), jnp.int32),
            pltpu.VMEM((CHUNK, d), jnp.float32),
            pltpu.VMEM_SHARED((NP, d), jnp.float32),
            pltpu.SemaphoreType.DMA,
        ],
        compiler_params=pltpu.CompilerParams(use_tc_tiling_on_sc=False),
    )
    def k(table_hbm, src_hbm, dst_hbm, zeros_hbm, out_hbm,
          sidx, didx, rows, acc, sem):
        cid = lax.axis_index("c")
        sid = lax.axis_index("s")
        wid = sid * NC + cid
        r0 = sid * ROWS_PER_TILE
        pltpu.sync_copy(zeros_hbm.at[pl.ds(r0, ROWS_PER_TILE)],
                        acc.at[pl.ds(r0, ROWS_PER_TILE)])
        plsc.subcore_barrier()

        trips = BASE_TRIPS + jnp.where(wid < EXTRA, 1, 0)

        def body(t, carry):
            base = (wid + t * NW) * CHUNK
            pltpu.sync_copy(src_hbm.at[pl.ds(base, CHUNK)], sidx)
            pltpu.sync_copy(dst_hbm.at[pl.ds(base, CHUNK)], didx)
            pltpu.async_copy(table_hbm.at[sidx], rows, sem).wait()
            pltpu.sync_copy(rows, acc.at[didx], add=True)
            return carry

        lax.fori_loop(0, trips, body, 0)
        plsc.subcore_barrier()
        pltpu.sync_copy(acc.at[pl.ds(r0, ROWS_PER_TILE)],
                        out_hbm.at[cid, pl.ds(r0, ROWS_PER_TILE)])

    return k


_sc_scatter_1 = _make_sc_scatter(D1)
_sc_scatter_2 = _make_sc_scatter(D2)


def _mm1_body(x_ref, w_ref, o_ref):
    y = jnp.dot(x_ref[...], w_ref[...], preferred_element_type=jnp.float32)
    ones = jnp.ones((y.shape[0], 1), jnp.float32)
    zeros = jnp.zeros((y.shape[0], D1 - 101), jnp.float32)
    o_ref[...] = jnp.concatenate([y, ones, zeros], axis=1)


def _mid_body(acc_ref, y_ref, b1_ref, w2_ref, o_ref):
    s = acc_ref[0] + acc_ref[1]
    deg = s[:, 100:101]
    agg = jnp.where(deg > 0.0,
                    s[:, :100] / jnp.maximum(deg, 1.0),
                    y_ref[:, :100])
    h1 = jnp.maximum(agg + b1_ref[...], 0.0)
    y2 = jnp.dot(h1, w2_ref[...], preferred_element_type=jnp.float32)
    ones = jnp.ones((y2.shape[0], 1), jnp.float32)
    zeros = jnp.zeros((y2.shape[0], D2 - 21), jnp.float32)
    o_ref[...] = jnp.concatenate([y2, ones, zeros], axis=1)


def _fin_body(acc_ref, y_ref, b2_ref, sf_ref, wf1_ref, bf1_ref, wf2_ref,
              bf2_ref, o_ref, scr):
    i = pl.program_id(0)

    @pl.when(i == 0)
    def _():
        scr[...] = jnp.zeros_like(scr)

    s = acc_ref[0] + acc_ref[1]
    deg = s[:, 20:21]
    agg = jnp.where(deg > 0.0,
                    s[:, :20] / jnp.maximum(deg, 1.0),
                    y_ref[:, :20])
    h2 = jnp.maximum(agg + b2_ref[...], 0.0)
    scr[...] += jnp.sum(h2, axis=0, keepdims=True)

    @pl.when(i == pl.num_programs(0) - 1)
    def _():
        hg = scr[...] / jnp.float32(N)
        z = (jnp.dot(hg, wf1_ref[0:20, :], preferred_element_type=jnp.float32)
             + jnp.dot(sf_ref[...], wf1_ref[20:36, :],
                       preferred_element_type=jnp.float32)
             + bf1_ref[...])
        z = jnp.maximum(z, 0.0)
        o_ref[...] = (jnp.dot(z, wf2_ref[...],
                              preferred_element_type=jnp.float32)
                      + bf2_ref[...])


def kernel(x, edge_index, self_feat, W1, b1, W2, b2, Wf1, bf1, Wf2, bf2):
    src = edge_index[0]
    dst = edge_index[1]
    zeros1 = jnp.zeros((NP, D1), jnp.float32)
    zeros2 = jnp.zeros((NP, D2), jnp.float32)

    y1p = pl.pallas_call(
        _mm1_body,
        grid=(N // BN,),
        in_specs=[
            pl.BlockSpec((BN, 128), lambda i: (i, 0)),
            pl.BlockSpec((128, 100), lambda i: (0, 0)),
        ],
        out_specs=pl.BlockSpec((BN, D1), lambda i: (i, 0)),
        out_shape=jax.ShapeDtypeStruct((N, D1), jnp.float32),
    )(x, W1)

    acc1 = _sc_scatter_1(y1p, src, dst, zeros1)

    y2p = pl.pallas_call(
        _mid_body,
        grid=(N // BN,),
        in_specs=[
            pl.BlockSpec((NC, BN, D1), lambda i: (0, i, 0)),
            pl.BlockSpec((BN, D1), lambda i: (i, 0)),
            pl.BlockSpec((1, 100), lambda i: (0, 0)),
            pl.BlockSpec((100, 20), lambda i: (0, 0)),
        ],
        out_specs=pl.BlockSpec((BN, D2), lambda i: (i, 0)),
        out_shape=jax.ShapeDtypeStruct((N, D2), jnp.float32),
    )(acc1, y1p, b1.reshape(1, 100), W2)

    acc2 = _sc_scatter_2(y2p, src, dst, zeros2)

    out = pl.pallas_call(
        _fin_body,
        grid=(N // BN,),
        in_specs=[
            pl.BlockSpec((NC, BN, D2), lambda i: (0, i, 0)),
            pl.BlockSpec((BN, D2), lambda i: (i, 0)),
            pl.BlockSpec((1, 20), lambda i: (0, 0)),
            pl.BlockSpec((1, 16), lambda i: (0, 0)),
            pl.BlockSpec((36, 10), lambda i: (0, 0)),
            pl.BlockSpec((1, 10), lambda i: (0, 0)),
            pl.BlockSpec((10, 10), lambda i: (0, 0)),
            pl.BlockSpec((1, 10), lambda i: (0, 0)),
        ],
        out_specs=pl.BlockSpec((1, 10), lambda i: (0, 0)),
        out_shape=jax.ShapeDtypeStruct((1, 10), jnp.float32),
        scratch_shapes=[pltpu.VMEM((1, 20), jnp.float32)],
    )(acc2, y2p, b2.reshape(1, 20), self_feat, Wf1, bf1.reshape(1, 10),
      Wf2, bf2.reshape(1, 10))

    return out


# speedup vs baseline: 15.6521x; 2.0019x over previous
import functools

import jax
import jax.numpy as jnp
from jax import lax
from jax.experimental import pallas as pl
from jax.experimental.pallas import tpu as pltpu
from jax.experimental.pallas import tpu_sc as plsc

N = 10000
E = 320000
D1 = 112
D2 = 32
NC = 2
NS = 16
NW = NC * NS
CHUNK = 100
NCH_W = E // (CHUNK * NW)
NP = 10112
ROWS_PER_TILE = NP // NS
RING = 3
SKEW = 1
BN = 1000


def _make_sc_scatter(d):
    mesh = plsc.VectorSubcoreMesh(core_axis_name="c", subcore_axis_name="s")

    @functools.partial(
        pl.kernel,
        mesh=mesh,
        out_type=jax.ShapeDtypeStruct((NC, NP, d), jnp.float32),
        scratch_types=[
            pltpu.VMEM((NCH_W, CHUNK), jnp.int32),
            pltpu.VMEM((NCH_W, CHUNK), jnp.int32),
            pltpu.VMEM((RING, CHUNK, d), jnp.float32),
            pltpu.VMEM_SHARED((NP, d), jnp.float32),
            pltpu.SemaphoreType.DMA((RING,)),
            pltpu.SemaphoreType.DMA((RING,)),
        ],
        compiler_params=pltpu.CompilerParams(use_tc_tiling_on_sc=False),
    )
    def k(table_hbm, src_hbm, dst_hbm, zeros_hbm, out_hbm,
          sidx, didx, rows, acc, gsem, ssem):
        cid = lax.axis_index("c")
        sid = lax.axis_index("s")
        wid = sid * NC + cid
        r0 = sid * ROWS_PER_TILE
        pltpu.sync_copy(zeros_hbm.at[pl.ds(r0, ROWS_PER_TILE)],
                        acc.at[pl.ds(r0, ROWS_PER_TILE)])
        pltpu.sync_copy(src_hbm.at[pl.ds(wid * NCH_W, NCH_W)], sidx)
        pltpu.sync_copy(dst_hbm.at[pl.ds(wid * NCH_W, NCH_W)], didx)
        plsc.subcore_barrier()

        def body(j, carry):
            r = j % RING

            @pl.when(j >= RING)
            def _():
                pltpu.make_async_copy(
                    rows.at[r], acc.at[didx.at[j - RING]], ssem.at[r]).wait()

            @pl.when(j < NCH_W)
            def _():
                pltpu.async_copy(table_hbm.at[sidx.at[j]], rows.at[r],
                                 gsem.at[r])

            @pl.when(j >= SKEW)
            def _():
                c = j - SKEW
                rc = c % RING
                pltpu.make_async_copy(table_hbm.at[sidx.at[c]], rows.at[rc],
                                      gsem.at[rc]).wait()
                pltpu.async_copy(rows.at[rc], acc.at[didx.at[c]],
                                 ssem.at[rc], add=True)

            return carry

        lax.fori_loop(0, NCH_W + SKEW, body, 0)
        for t in range(RING - SKEW):
            c = NCH_W - (RING - SKEW) + t
            rc = c % RING
            pltpu.make_async_copy(rows.at[rc], acc.at[didx.at[c]],
                                  ssem.at[rc]).wait()
        plsc.subcore_barrier()
        pltpu.sync_copy(acc.at[pl.ds(r0, ROWS_PER_TILE)],
                        out_hbm.at[cid, pl.ds(r0, ROWS_PER_TILE)])

    return k


_sc_scatter_1 = _make_sc_scatter(D1)
_sc_scatter_2 = _make_sc_scatter(D2)


def _mm1_body(x_ref, w_ref, o_ref):
    y = jnp.dot(x_ref[...], w_ref[...], preferred_element_type=jnp.float32)
    ones = jnp.ones((y.shape[0], 1), jnp.float32)
    zeros = jnp.zeros((y.shape[0], D1 - 101), jnp.float32)
    o_ref[...] = jnp.concatenate([y, ones, zeros], axis=1)


def _mid_body(acc_ref, y_ref, b1_ref, w2_ref, o_ref):
    s = acc_ref[0] + acc_ref[1]
    deg = s[:, 100:101]
    agg = jnp.where(deg > 0.0,
                    s[:, :100] / jnp.maximum(deg, 1.0),
                    y_ref[:, :100])
    h1 = jnp.maximum(agg + b1_ref[...], 0.0)
    y2 = jnp.dot(h1, w2_ref[...], preferred_element_type=jnp.float32)
    ones = jnp.ones((y2.shape[0], 1), jnp.float32)
    zeros = jnp.zeros((y2.shape[0], D2 - 21), jnp.float32)
    o_ref[...] = jnp.concatenate([y2, ones, zeros], axis=1)


def _fin_body(acc_ref, y_ref, b2_ref, sf_ref, wf1_ref, bf1_ref, wf2_ref,
              bf2_ref, o_ref, scr):
    i = pl.program_id(0)

    @pl.when(i == 0)
    def _():
        scr[...] = jnp.zeros_like(scr)

    s = acc_ref[0] + acc_ref[1]
    deg = s[:, 20:21]
    agg = jnp.where(deg > 0.0,
                    s[:, :20] / jnp.maximum(deg, 1.0),
                    y_ref[:, :20])
    h2 = jnp.maximum(agg + b2_ref[...], 0.0)
    scr[...] += jnp.sum(h2, axis=0, keepdims=True)

    @pl.when(i == pl.num_programs(0) - 1)
    def _():
        hg = scr[...] / jnp.float32(N)
        z = (jnp.dot(hg, wf1_ref[0:20, :], preferred_element_type=jnp.float32)
             + jnp.dot(sf_ref[...], wf1_ref[20:36, :],
                       preferred_element_type=jnp.float32)
             + bf1_ref[...])
        z = jnp.maximum(z, 0.0)
        o_ref[...] = (jnp.dot(z, wf2_ref[...],
                              preferred_element_type=jnp.float32)
                      + bf2_ref[...])


def kernel(x, edge_index, self_feat, W1, b1, W2, b2, Wf1, bf1, Wf2, bf2):
    src = edge_index[0].reshape(NW * NCH_W, CHUNK)
    dst = edge_index[1].reshape(NW * NCH_W, CHUNK)
    zeros1 = jnp.zeros((NP, D1), jnp.float32)
    zeros2 = jnp.zeros((NP, D2), jnp.float32)

    y1p = pl.pallas_call(
        _mm1_body,
        grid=(N // BN,),
        in_specs=[
            pl.BlockSpec((BN, 128), lambda i: (i, 0)),
            pl.BlockSpec((128, 100), lambda i: (0, 0)),
        ],
        out_specs=pl.BlockSpec((BN, D1), lambda i: (i, 0)),
        out_shape=jax.ShapeDtypeStruct((N, D1), jnp.float32),
    )(x, W1)

    acc1 = _sc_scatter_1(y1p, src, dst, zeros1)

    y2p = pl.pallas_call(
        _mid_body,
        grid=(N // BN,),
        in_specs=[
            pl.BlockSpec((NC, BN, D1), lambda i: (0, i, 0)),
            pl.BlockSpec((BN, D1), lambda i: (i, 0)),
            pl.BlockSpec((1, 100), lambda i: (0, 0)),
            pl.BlockSpec((100, 20), lambda i: (0, 0)),
        ],
        out_specs=pl.BlockSpec((BN, D2), lambda i: (i, 0)),
        out_shape=jax.ShapeDtypeStruct((N, D2), jnp.float32),
    )(acc1, y1p, b1.reshape(1, 100), W2)

    acc2 = _sc_scatter_2(y2p, src, dst, zeros2)

    out = pl.pallas_call(
        _fin_body,
        grid=(N // BN,),
        in_specs=[
            pl.BlockSpec((NC, BN, D2), lambda i: (0, i, 0)),
            pl.BlockSpec((BN, D2), lambda i: (i, 0)),
            pl.BlockSpec((1, 20), lambda i: (0, 0)),
            pl.BlockSpec((1, 16), lambda i: (0, 0)),
            pl.BlockSpec((36, 10), lambda i: (0, 0)),
            pl.BlockSpec((1, 10), lambda i: (0, 0)),
            pl.BlockSpec((10, 10), lambda i: (0, 0)),
            pl.BlockSpec((1, 10), lambda i: (0, 0)),
        ],
        out_specs=pl.BlockSpec((1, 10), lambda i: (0, 0)),
        out_shape=jax.ShapeDtypeStruct((1, 10), jnp.float32),
        scratch_shapes=[pltpu.VMEM((1, 20), jnp.float32)],
    )(acc2, y2p, b2.reshape(1, 20), self_feat, Wf1, bf1.reshape(1, 10),
      Wf2, bf2.reshape(1, 10))

    return out


# speedup vs baseline: 16.2592x; 1.0388x over previous
import functools

import jax
import jax.numpy as jnp
from jax import lax
from jax.experimental import pallas as pl
from jax.experimental.pallas import tpu as pltpu
from jax.experimental.pallas import tpu_sc as plsc

N = 10000
E = 320000
D1 = 128
D2 = 32
NC = 2
NS = 16
NW = NC * NS
CHUNK = 100
NCH_W = E // (CHUNK * NW)
BLK = 20
NBLK = NCH_W // BLK
NP = 10112
ROWS_PER_TILE = NP // NS
RING = 3
SKEW = 1
BN = 1000


def _make_sc_scatter(d):
    mesh = plsc.VectorSubcoreMesh(core_axis_name="c", subcore_axis_name="s")
    nzero = CHUNK * d // 16
    lanes_per_row = d // 16
    n_full, rem = divmod(ROWS_PER_TILE, CHUNK)

    @functools.partial(
        pl.kernel,
        mesh=mesh,
        out_type=jax.ShapeDtypeStruct((NC, NP, d), jnp.float32),
        scratch_types=[
            pltpu.VMEM((2, BLK, CHUNK), jnp.int32),
            pltpu.VMEM((2, BLK, CHUNK), jnp.int32),
            pltpu.VMEM((RING, CHUNK, d), jnp.float32),
            pltpu.VMEM_SHARED((NP, d), jnp.float32),
            pltpu.SemaphoreType.DMA((RING,)),
            pltpu.SemaphoreType.DMA((RING,)),
        ],
        compiler_params=pltpu.CompilerParams(use_tc_tiling_on_sc=False),
    )
    def k(table_hbm, edge_hbm, out_hbm, sidx, didx, rows, acc, gsem, ssem):
        cid = lax.axis_index("c")
        sid = lax.axis_index("s")
        wid = sid * NC + cid
        r0 = sid * ROWS_PER_TILE
        base = wid * NCH_W

        def zbody(i, carry):
            rr = i // lanes_per_row
            cc = (i % lanes_per_row) * 16
            rows[0, rr, pl.ds(cc, 16)] = jnp.zeros((16,), jnp.float32)
            return carry

        lax.fori_loop(0, nzero, zbody, 0)
        for kk in range(n_full):
            pltpu.sync_copy(rows.at[0],
                            acc.at[pl.ds(r0 + kk * CHUNK, CHUNK)])
        if rem:
            pltpu.sync_copy(rows.at[0, pl.ds(0, rem)],
                            acc.at[pl.ds(r0 + n_full * CHUNK, rem)])

        for b in range(2):
            pltpu.sync_copy(edge_hbm.at[0, pl.ds(base + b * BLK, BLK)],
                            sidx.at[b])
            pltpu.sync_copy(edge_hbm.at[1, pl.ds(base + b * BLK, BLK)],
                            didx.at[b])
        plsc.subcore_barrier()

        def srow(c):
            return sidx.at[(c // BLK) % 2, c % BLK]

        def drow(c):
            return didx.at[(c // BLK) % 2, c % BLK]

        def body(j, carry):
            r = j % RING

            @pl.when((j % BLK == 5) & (j >= BLK + 5)
                     & (j <= (NBLK - 2) * BLK + 5))
            def _():
                b = j // BLK + 1
                slot = b % 2
                pltpu.sync_copy(edge_hbm.at[0, pl.ds(base + b * BLK, BLK)],
                                sidx.at[slot])
                pltpu.sync_copy(edge_hbm.at[1, pl.ds(base + b * BLK, BLK)],
                                didx.at[slot])

            @pl.when(j >= RING)
            def _():
                pltpu.make_async_copy(
                    rows.at[r], acc.at[drow(j - RING)], ssem.at[r]).wait()

            @pl.when(j < NCH_W)
            def _():
                pltpu.async_copy(table_hbm.at[srow(j)], rows.at[r],
                                 gsem.at[r])

            @pl.when(j >= SKEW)
            def _():
                c = j - SKEW
                rc = c % RING
                pltpu.make_async_copy(table_hbm.at[srow(c)], rows.at[rc],
                                      gsem.at[rc]).wait()
                pltpu.async_copy(rows.at[rc], acc.at[drow(c)],
                                 ssem.at[rc], add=True)

            return carry

        lax.fori_loop(0, NCH_W + SKEW, body, 0)
        for t in range(RING - SKEW):
            c = NCH_W - (RING - SKEW) + t
            rc = c % RING
            pltpu.make_async_copy(rows.at[rc], acc.at[drow(c)],
                                  ssem.at[rc]).wait()
        plsc.subcore_barrier()
        pltpu.sync_copy(acc.at[pl.ds(r0, ROWS_PER_TILE)],
                        out_hbm.at[cid, pl.ds(r0, ROWS_PER_TILE)])

    return k


_sc_scatter_1 = _make_sc_scatter(D1)
_sc_scatter_2 = _make_sc_scatter(D2)


def _mm1_body(x_ref, w_ref, o_ref):
    y = jnp.dot(x_ref[...], w_ref[...], preferred_element_type=jnp.float32)
    ones = jnp.ones((y.shape[0], 1), jnp.float32)
    zeros = jnp.zeros((y.shape[0], D1 - 101), jnp.float32)
    o_ref[...] = jnp.concatenate([y, ones, zeros], axis=1)


def _mid_body(acc_ref, y_ref, b1_ref, w2_ref, o_ref):
    s = acc_ref[0] + acc_ref[1]
    deg = s[:, 100:101]
    agg = jnp.where(deg > 0.0,
                    s[:, :100] / jnp.maximum(deg, 1.0),
                    y_ref[:, :100])
    h1 = jnp.maximum(agg + b1_ref[...], 0.0)
    y2 = jnp.dot(h1, w2_ref[...], preferred_element_type=jnp.float32)
    ones = jnp.ones((y2.shape[0], 1), jnp.float32)
    zeros = jnp.zeros((y2.shape[0], D2 - 21), jnp.float32)
    o_ref[...] = jnp.concatenate([y2, ones, zeros], axis=1)


def _fin_body(acc_ref, y_ref, b2_ref, sf_ref, wf1_ref, bf1_ref, wf2_ref,
              bf2_ref, o_ref, scr):
    i = pl.program_id(0)

    @pl.when(i == 0)
    def _():
        scr[...] = jnp.zeros_like(scr)

    s = acc_ref[0] + acc_ref[1]
    deg = s[:, 20:21]
    agg = jnp.where(deg > 0.0,
                    s[:, :20] / jnp.maximum(deg, 1.0),
                    y_ref[:, :20])
    h2 = jnp.maximum(agg + b2_ref[...], 0.0)
    scr[...] += jnp.sum(h2, axis=0, keepdims=True)

    @pl.when(i == pl.num_programs(0) - 1)
    def _():
        hg = scr[...] / jnp.float32(N)
        z = (jnp.dot(hg, wf1_ref[0:20, :], preferred_element_type=jnp.float32)
             + jnp.dot(sf_ref[...], wf1_ref[20:36, :],
                       preferred_element_type=jnp.float32)
             + bf1_ref[...])
        z = jnp.maximum(z, 0.0)
        o_ref[...] = (jnp.dot(z, wf2_ref[...],
                              preferred_element_type=jnp.float32)
                      + bf2_ref[...])


def kernel(x, edge_index, self_feat, W1, b1, W2, b2, Wf1, bf1, Wf2, bf2):
    edges = edge_index.reshape(2, NW * NCH_W, CHUNK)

    y1p = pl.pallas_call(
        _mm1_body,
        grid=(N // BN,),
        in_specs=[
            pl.BlockSpec((BN, 128), lambda i: (i, 0)),
            pl.BlockSpec((128, 100), lambda i: (0, 0)),
        ],
        out_specs=pl.BlockSpec((BN, D1), lambda i: (i, 0)),
        out_shape=jax.ShapeDtypeStruct((NP, D1), jnp.float32),
    )(x, W1)

    acc1 = _sc_scatter_1(y1p, edges)

    y2p = pl.pallas_call(
        _mid_body,
        grid=(N // BN,),
        in_specs=[
            pl.BlockSpec((NC, BN, D1), lambda i: (0, i, 0)),
            pl.BlockSpec((BN, D1), lambda i: (i, 0)),
            pl.BlockSpec((1, 100), lambda i: (0, 0)),
            pl.BlockSpec((100, 20), lambda i: (0, 0)),
        ],
        out_specs=pl.BlockSpec((BN, D2), lambda i: (i, 0)),
        out_shape=jax.ShapeDtypeStruct((NP, D2), jnp.float32),
    )(acc1, y1p, b1.reshape(1, 100), W2)

    acc2 = _sc_scatter_2(y2p, edges)

    out = pl.pallas_call(
        _fin_body,
        grid=(N // BN,),
        in_specs=[
            pl.BlockSpec((NC, BN, D2), lambda i: (0, i, 0)),
            pl.BlockSpec((BN, D2), lambda i: (i, 0)),
            pl.BlockSpec((1, 20), lambda i: (0, 0)),
            pl.BlockSpec((1, 16), lambda i: (0, 0)),
            pl.BlockSpec((36, 10), lambda i: (0, 0)),
            pl.BlockSpec((1, 10), lambda i: (0, 0)),
            pl.BlockSpec((10, 10), lambda i: (0, 0)),
            pl.BlockSpec((1, 10), lambda i: (0, 0)),
        ],
        out_specs=pl.BlockSpec((1, 10), lambda i: (0, 0)),
        out_shape=jax.ShapeDtypeStruct((1, 10), jnp.float32),
        scratch_shapes=[pltpu.VMEM((1, 20), jnp.float32)],
    )(acc2, y2p, b2.reshape(1, 20), self_feat, Wf1, bf1.reshape(1, 10),
      Wf2, bf2.reshape(1, 10))

    return out


# speedup vs baseline: 17.6297x; 1.0843x over previous
import functools

import jax
import jax.numpy as jnp
from jax import lax
from jax.experimental import pallas as pl
from jax.experimental.pallas import tpu as pltpu
from jax.experimental.pallas import tpu_sc as plsc

N = 10000
E = 320000
D1 = 128
D2 = 32
NC = 2
NS = 16
NW = NC * NS
CHUNK = 80
NCH_W = E // (CHUNK * NW)
BLK = 25
NBLK = NCH_W // BLK
NP = 10112
ROWS_PER_TILE = NP // NS
RING = 4
SKEW = 2
BN = 1000


def _make_sc_scatter(d):
    mesh = plsc.VectorSubcoreMesh(core_axis_name="c", subcore_axis_name="s")
    nzero = CHUNK * d // 16
    lanes_per_row = d // 16
    n_full, rem = divmod(ROWS_PER_TILE, CHUNK)

    @functools.partial(
        pl.kernel,
        mesh=mesh,
        out_type=jax.ShapeDtypeStruct((NC, NP, d), jnp.float32),
        scratch_types=[
            pltpu.VMEM((2, BLK, CHUNK), jnp.int32),
            pltpu.VMEM((2, BLK, CHUNK), jnp.int32),
            pltpu.VMEM((RING, CHUNK, d), jnp.float32),
            pltpu.VMEM_SHARED((NP, d), jnp.float32),
            pltpu.SemaphoreType.DMA((RING,)),
            pltpu.SemaphoreType.DMA((RING,)),
        ],
        compiler_params=pltpu.CompilerParams(use_tc_tiling_on_sc=False),
    )
    def k(table_hbm, edge_hbm, out_hbm, sidx, didx, rows, acc, gsem, ssem):
        cid = lax.axis_index("c")
        sid = lax.axis_index("s")
        wid = sid * NC + cid
        r0 = sid * ROWS_PER_TILE
        base = wid * NCH_W

        def zbody(i, carry):
            rr = i // lanes_per_row
            cc = (i % lanes_per_row) * 16
            rows[0, rr, pl.ds(cc, 16)] = jnp.zeros((16,), jnp.float32)
            return carry

        lax.fori_loop(0, nzero, zbody, 0)
        for kk in range(n_full):
            pltpu.sync_copy(rows.at[0],
                            acc.at[pl.ds(r0 + kk * CHUNK, CHUNK)])
        if rem:
            pltpu.sync_copy(rows.at[0, pl.ds(0, rem)],
                            acc.at[pl.ds(r0 + n_full * CHUNK, rem)])

        for b in range(2):
            pltpu.sync_copy(edge_hbm.at[0, pl.ds(base + b * BLK, BLK)],
                            sidx.at[b])
            pltpu.sync_copy(edge_hbm.at[1, pl.ds(base + b * BLK, BLK)],
                            didx.at[b])
        plsc.subcore_barrier()

        def srow(c):
            return sidx.at[(c // BLK) % 2, c % BLK]

        def drow(c):
            return didx.at[(c // BLK) % 2, c % BLK]

        def body(j, carry):
            r = j % RING

            @pl.when((j % BLK == 5) & (j >= BLK + 5)
                     & (j <= (NBLK - 2) * BLK + 5))
            def _():
                b = j // BLK + 1
                slot = b % 2
                pltpu.sync_copy(edge_hbm.at[0, pl.ds(base + b * BLK, BLK)],
                                sidx.at[slot])
                pltpu.sync_copy(edge_hbm.at[1, pl.ds(base + b * BLK, BLK)],
                                didx.at[slot])

            @pl.when(j >= RING)
            def _():
                pltpu.make_async_copy(
                    rows.at[r], acc.at[drow(j - RING)], ssem.at[r]).wait()

            @pl.when(j < NCH_W)
            def _():
                pltpu.async_copy(table_hbm.at[srow(j)], rows.at[r],
                                 gsem.at[r])

            @pl.when(j >= SKEW)
            def _():
                c = j - SKEW
                rc = c % RING
                pltpu.make_async_copy(table_hbm.at[srow(c)], rows.at[rc],
                                      gsem.at[rc]).wait()
                pltpu.async_copy(rows.at[rc], acc.at[drow(c)],
                                 ssem.at[rc], add=True)

            return carry

        lax.fori_loop(0, NCH_W + SKEW, body, 0)
        for t in range(RING - SKEW):
            c = NCH_W - (RING - SKEW) + t
            rc = c % RING
            pltpu.make_async_copy(rows.at[rc], acc.at[drow(c)],
                                  ssem.at[rc]).wait()
        plsc.subcore_barrier()
        pltpu.sync_copy(acc.at[pl.ds(r0, ROWS_PER_TILE)],
                        out_hbm.at[cid, pl.ds(r0, ROWS_PER_TILE)])

    return k


_sc_scatter_1 = _make_sc_scatter(D1)
_sc_scatter_2 = _make_sc_scatter(D2)


def _mm1_body(x_ref, w_ref, o_ref):
    y = jnp.dot(x_ref[...], w_ref[...], preferred_element_type=jnp.float32)
    ones = jnp.ones((y.shape[0], 1), jnp.float32)
    zeros = jnp.zeros((y.shape[0], D1 - 101), jnp.float32)
    o_ref[...] = jnp.concatenate([y, ones, zeros], axis=1)


def _mid_body(acc_ref, y_ref, b1_ref, w2_ref, o_ref):
    s = acc_ref[0] + acc_ref[1]
    deg = s[:, 100:101]
    agg = jnp.where(deg > 0.0,
                    s[:, :100] / jnp.maximum(deg, 1.0),
                    y_ref[:, :100])
    h1 = jnp.maximum(agg + b1_ref[...], 0.0)
    y2 = jnp.dot(h1, w2_ref[...], preferred_element_type=jnp.float32)
    ones = jnp.ones((y2.shape[0], 1), jnp.float32)
    zeros = jnp.zeros((y2.shape[0], D2 - 21), jnp.float32)
    o_ref[...] = jnp.concatenate([y2, ones, zeros], axis=1)


def _fin_body(acc_ref, y_ref, b2_ref, sf_ref, wf1_ref, bf1_ref, wf2_ref,
              bf2_ref, o_ref, scr):
    i = pl.program_id(0)

    @pl.when(i == 0)
    def _():
        scr[...] = jnp.zeros_like(scr)

    s = acc_ref[0] + acc_ref[1]
    deg = s[:, 20:21]
    agg = jnp.where(deg > 0.0,
                    s[:, :20] / jnp.maximum(deg, 1.0),
                    y_ref[:, :20])
    h2 = jnp.maximum(agg + b2_ref[...], 0.0)
    scr[...] += jnp.sum(h2, axis=0, keepdims=True)

    @pl.when(i == pl.num_programs(0) - 1)
    def _():
        hg = scr[...] / jnp.float32(N)
        z = (jnp.dot(hg, wf1_ref[0:20, :], preferred_element_type=jnp.float32)
             + jnp.dot(sf_ref[...], wf1_ref[20:36, :],
                       preferred_element_type=jnp.float32)
             + bf1_ref[...])
        z = jnp.maximum(z, 0.0)
        o_ref[...] = (jnp.dot(z, wf2_ref[...],
                              preferred_element_type=jnp.float32)
                      + bf2_ref[...])


def kernel(x, edge_index, self_feat, W1, b1, W2, b2, Wf1, bf1, Wf2, bf2):
    edges = edge_index.reshape(2, NW * NCH_W, CHUNK)

    y1p = pl.pallas_call(
        _mm1_body,
        grid=(N // BN,),
        in_specs=[
            pl.BlockSpec((BN, 128), lambda i: (i, 0)),
            pl.BlockSpec((128, 100), lambda i: (0, 0)),
        ],
        out_specs=pl.BlockSpec((BN, D1), lambda i: (i, 0)),
        out_shape=jax.ShapeDtypeStruct((NP, D1), jnp.float32),
    )(x, W1)

    acc1 = _sc_scatter_1(y1p, edges)

    y2p = pl.pallas_call(
        _mid_body,
        grid=(N // BN,),
        in_specs=[
            pl.BlockSpec((NC, BN, D1), lambda i: (0, i, 0)),
            pl.BlockSpec((BN, D1), lambda i: (i, 0)),
            pl.BlockSpec((1, 100), lambda i: (0, 0)),
            pl.BlockSpec((100, 20), lambda i: (0, 0)),
        ],
        out_specs=pl.BlockSpec((BN, D2), lambda i: (i, 0)),
        out_shape=jax.ShapeDtypeStruct((NP, D2), jnp.float32),
    )(acc1, y1p, b1.reshape(1, 100), W2)

    acc2 = _sc_scatter_2(y2p, edges)

    out = pl.pallas_call(
        _fin_body,
        grid=(N // BN,),
        in_specs=[
            pl.BlockSpec((NC, BN, D2), lambda i: (0, i, 0)),
            pl.BlockSpec((BN, D2), lambda i: (i, 0)),
            pl.BlockSpec((1, 20), lambda i: (0, 0)),
            pl.BlockSpec((1, 16), lambda i: (0, 0)),
            pl.BlockSpec((36, 10), lambda i: (0, 0)),
            pl.BlockSpec((1, 10), lambda i: (0, 0)),
            pl.BlockSpec((10, 10), lambda i: (0, 0)),
            pl.BlockSpec((1, 10), lambda i: (0, 0)),
        ],
        out_specs=pl.BlockSpec((1, 10), lambda i: (0, 0)),
        out_shape=jax.ShapeDtypeStruct((1, 10), jnp.float32),
        scratch_shapes=[pltpu.VMEM((1, 20), jnp.float32)],
    )(acc2, y2p, b2.reshape(1, 20), self_feat, Wf1, bf1.reshape(1, 10),
      Wf2, bf2.reshape(1, 10))

    return out


# speedup vs baseline: 19.2913x; 1.0942x over previous
import functools

import jax
import jax.numpy as jnp
from jax import lax
from jax.experimental import pallas as pl
from jax.experimental.pallas import tpu as pltpu
from jax.experimental.pallas import tpu_sc as plsc

N = 10000
E = 320000
D1 = 128
D2 = 32
NC = 2
NS = 16
NW = NC * NS
CHUNK = 80
NCH_W = E // (CHUNK * NW)
BLK = 25
NBLK = NCH_W // BLK
NP = 10112
ROWS_PER_TILE = NP // NS
RING = 4
SKEW = 3
BN = 2000


def _make_sc_scatter(d):
    mesh = plsc.VectorSubcoreMesh(core_axis_name="c", subcore_axis_name="s")
    nzero = CHUNK * d // 16
    lanes_per_row = d // 16
    n_full, rem = divmod(ROWS_PER_TILE, CHUNK)

    @functools.partial(
        pl.kernel,
        mesh=mesh,
        out_type=jax.ShapeDtypeStruct((NC, NP, d), jnp.float32),
        scratch_types=[
            pltpu.VMEM((2, BLK, CHUNK), jnp.int32),
            pltpu.VMEM((2, BLK, CHUNK), jnp.int32),
            pltpu.VMEM((RING, CHUNK, d), jnp.float32),
            pltpu.VMEM_SHARED((NP, d), jnp.float32),
            pltpu.SemaphoreType.DMA((RING,)),
            pltpu.SemaphoreType.DMA((RING,)),
        ],
        compiler_params=pltpu.CompilerParams(use_tc_tiling_on_sc=False),
    )
    def k(table_hbm, edge_hbm, out_hbm, sidx, didx, rows, acc, gsem, ssem):
        cid = lax.axis_index("c")
        sid = lax.axis_index("s")
        wid = sid * NC + cid
        r0 = sid * ROWS_PER_TILE
        base = wid * NCH_W

        def zbody(i, carry):
            rr = i // lanes_per_row
            cc = (i % lanes_per_row) * 16
            rows[0, rr, pl.ds(cc, 16)] = jnp.zeros((16,), jnp.float32)
            return carry

        lax.fori_loop(0, nzero, zbody, 0)
        for kk in range(n_full):
            pltpu.sync_copy(rows.at[0],
                            acc.at[pl.ds(r0 + kk * CHUNK, CHUNK)])
        if rem:
            pltpu.sync_copy(rows.at[0, pl.ds(0, rem)],
                            acc.at[pl.ds(r0 + n_full * CHUNK, rem)])

        for b in range(2):
            pltpu.sync_copy(edge_hbm.at[0, pl.ds(base + b * BLK, BLK)],
                            sidx.at[b])
            pltpu.sync_copy(edge_hbm.at[1, pl.ds(base + b * BLK, BLK)],
                            didx.at[b])
        plsc.subcore_barrier()

        def srow(c):
            return sidx.at[(c // BLK) % 2, c % BLK]

        def drow(c):
            return didx.at[(c // BLK) % 2, c % BLK]

        def body(j, carry):
            r = j % RING

            @pl.when((j % BLK == 5) & (j >= BLK + 5)
                     & (j <= (NBLK - 2) * BLK + 5))
            def _():
                b = j // BLK + 1
                slot = b % 2
                pltpu.sync_copy(edge_hbm.at[0, pl.ds(base + b * BLK, BLK)],
                                sidx.at[slot])
                pltpu.sync_copy(edge_hbm.at[1, pl.ds(base + b * BLK, BLK)],
                                didx.at[slot])

            @pl.when(j >= RING)
            def _():
                pltpu.make_async_copy(
                    rows.at[r], acc.at[drow(j - RING)], ssem.at[r]).wait()

            @pl.when(j < NCH_W)
            def _():
                pltpu.async_copy(table_hbm.at[srow(j)], rows.at[r],
                                 gsem.at[r])

            @pl.when(j >= SKEW)
            def _():
                c = j - SKEW
                rc = c % RING
                pltpu.make_async_copy(table_hbm.at[srow(c)], rows.at[rc],
                                      gsem.at[rc]).wait()
                pltpu.async_copy(rows.at[rc], acc.at[drow(c)],
                                 ssem.at[rc], add=True)

            return carry

        lax.fori_loop(0, NCH_W + SKEW, body, 0)
        for t in range(RING - SKEW):
            c = NCH_W - (RING - SKEW) + t
            rc = c % RING
            pltpu.make_async_copy(rows.at[rc], acc.at[drow(c)],
                                  ssem.at[rc]).wait()
        plsc.subcore_barrier()
        pltpu.sync_copy(acc.at[pl.ds(r0, ROWS_PER_TILE)],
                        out_hbm.at[cid, pl.ds(r0, ROWS_PER_TILE)])

    return k


_sc_scatter_1 = _make_sc_scatter(D1)
_sc_scatter_2 = _make_sc_scatter(D2)


def _mm1_body(x_ref, w_ref, o_ref):
    y = jnp.dot(x_ref[...], w_ref[...], preferred_element_type=jnp.float32)
    ones = jnp.ones((y.shape[0], 1), jnp.float32)
    zeros = jnp.zeros((y.shape[0], D1 - 101), jnp.float32)
    o_ref[...] = jnp.concatenate([y, ones, zeros], axis=1)


def _mid_body(acc_ref, y_ref, b1_ref, w2_ref, o_ref):
    s = acc_ref[0] + acc_ref[1]
    deg = s[:, 100:101]
    agg = jnp.where(deg > 0.0,
                    s[:, :100] / jnp.maximum(deg, 1.0),
                    y_ref[:, :100])
    h1 = jnp.maximum(agg + b1_ref[...], 0.0)
    y2 = jnp.dot(h1, w2_ref[...], preferred_element_type=jnp.float32)
    ones = jnp.ones((y2.shape[0], 1), jnp.float32)
    zeros = jnp.zeros((y2.shape[0], D2 - 21), jnp.float32)
    o_ref[...] = jnp.concatenate([y2, ones, zeros], axis=1)


def _fin_body(acc_ref, y_ref, b2_ref, sf_ref, wf1_ref, bf1_ref, wf2_ref,
              bf2_ref, o_ref, scr):
    i = pl.program_id(0)

    @pl.when(i == 0)
    def _():
        scr[...] = jnp.zeros_like(scr)

    s = acc_ref[0] + acc_ref[1]
    deg = s[:, 20:21]
    agg = jnp.where(deg > 0.0,
                    s[:, :20] / jnp.maximum(deg, 1.0),
                    y_ref[:, :20])
    h2 = jnp.maximum(agg + b2_ref[...], 0.0)
    scr[...] += jnp.sum(h2, axis=0, keepdims=True)

    @pl.when(i == pl.num_programs(0) - 1)
    def _():
        hg = scr[...] / jnp.float32(N)
        z = (jnp.dot(hg, wf1_ref[0:20, :], preferred_element_type=jnp.float32)
             + jnp.dot(sf_ref[...], wf1_ref[20:36, :],
                       preferred_element_type=jnp.float32)
             + bf1_ref[...])
        z = jnp.maximum(z, 0.0)
        o_ref[...] = (jnp.dot(z, wf2_ref[...],
                              preferred_element_type=jnp.float32)
                      + bf2_ref[...])


def kernel(x, edge_index, self_feat, W1, b1, W2, b2, Wf1, bf1, Wf2, bf2):
    edges = edge_index.reshape(2, NW * NCH_W, CHUNK)

    y1p = pl.pallas_call(
        _mm1_body,
        grid=(N // BN,),
        in_specs=[
            pl.BlockSpec((BN, 128), lambda i: (i, 0)),
            pl.BlockSpec((128, 100), lambda i: (0, 0)),
        ],
        out_specs=pl.BlockSpec((BN, D1), lambda i: (i, 0)),
        out_shape=jax.ShapeDtypeStruct((NP, D1), jnp.float32),
    )(x, W1)

    acc1 = _sc_scatter_1(y1p, edges)

    y2p = pl.pallas_call(
        _mid_body,
        grid=(N // BN,),
        in_specs=[
            pl.BlockSpec((NC, BN, D1), lambda i: (0, i, 0)),
            pl.BlockSpec((BN, D1), lambda i: (i, 0)),
            pl.BlockSpec((1, 100), lambda i: (0, 0)),
            pl.BlockSpec((100, 20), lambda i: (0, 0)),
        ],
        out_specs=pl.BlockSpec((BN, D2), lambda i: (i, 0)),
        out_shape=jax.ShapeDtypeStruct((NP, D2), jnp.float32),
    )(acc1, y1p, b1.reshape(1, 100), W2)

    acc2 = _sc_scatter_2(y2p, edges)

    out = pl.pallas_call(
        _fin_body,
        grid=(N // BN,),
        in_specs=[
            pl.BlockSpec((NC, BN, D2), lambda i: (0, i, 0)),
            pl.BlockSpec((BN, D2), lambda i: (i, 0)),
            pl.BlockSpec((1, 20), lambda i: (0, 0)),
            pl.BlockSpec((1, 16), lambda i: (0, 0)),
            pl.BlockSpec((36, 10), lambda i: (0, 0)),
            pl.BlockSpec((1, 10), lambda i: (0, 0)),
            pl.BlockSpec((10, 10), lambda i: (0, 0)),
            pl.BlockSpec((1, 10), lambda i: (0, 0)),
        ],
        out_specs=pl.BlockSpec((1, 10), lambda i: (0, 0)),
        out_shape=jax.ShapeDtypeStruct((1, 10), jnp.float32),
        scratch_shapes=[pltpu.VMEM((1, 20), jnp.float32)],
    )(acc2, y2p, b2.reshape(1, 20), self_feat, Wf1, bf1.reshape(1, 10),
      Wf2, bf2.reshape(1, 10))

    return out


# speedup vs baseline: 19.7016x; 1.0213x over previous
import functools

import jax
import jax.numpy as jnp
from jax import lax
from jax.experimental import pallas as pl
from jax.experimental.pallas import tpu as pltpu
from jax.experimental.pallas import tpu_sc as plsc

N = 10000
E = 320000
D1 = 128
D2 = 32
NC = 2
NS = 16
NW = NC * NS
NP = 10112
ROWS_PER_TILE = NP // NS
RING = 4
SKEW = 3
BN = 2000
CHUNK1, BLK1 = 80, 25
CHUNK2, BLK2 = 125, 16


def _make_sc_scatter(d, CHUNK, BLK):
    mesh = plsc.VectorSubcoreMesh(core_axis_name="c", subcore_axis_name="s")
    NCH_W = E // (CHUNK * NW)
    NBLK = NCH_W // BLK
    nzero = CHUNK * d // 16
    lanes_per_row = d // 16
    n_full, rem = divmod(ROWS_PER_TILE, CHUNK)

    @functools.partial(
        pl.kernel,
        mesh=mesh,
        out_type=jax.ShapeDtypeStruct((NC, NP, d), jnp.float32),
        scratch_types=[
            pltpu.VMEM((2, BLK, CHUNK), jnp.int32),
            pltpu.VMEM((2, BLK, CHUNK), jnp.int32),
            pltpu.VMEM((RING, CHUNK, d), jnp.float32),
            pltpu.VMEM_SHARED((NP, d), jnp.float32),
            pltpu.SemaphoreType.DMA((RING,)),
            pltpu.SemaphoreType.DMA((RING,)),
        ],
        compiler_params=pltpu.CompilerParams(use_tc_tiling_on_sc=False),
    )
    def k(table_hbm, edge_hbm, out_hbm, sidx, didx, rows, acc, gsem, ssem):
        cid = lax.axis_index("c")
        sid = lax.axis_index("s")
        wid = sid * NC + cid
        r0 = sid * ROWS_PER_TILE
        base = wid * NCH_W

        def zbody(i, carry):
            rr = i // lanes_per_row
            cc = (i % lanes_per_row) * 16
            rows[0, rr, pl.ds(cc, 16)] = jnp.zeros((16,), jnp.float32)
            return carry

        lax.fori_loop(0, nzero, zbody, 0)
        for kk in range(n_full):
            pltpu.sync_copy(rows.at[0],
                            acc.at[pl.ds(r0 + kk * CHUNK, CHUNK)])
        if rem:
            pltpu.sync_copy(rows.at[0, pl.ds(0, rem)],
                            acc.at[pl.ds(r0 + n_full * CHUNK, rem)])

        for b in range(2):
            pltpu.sync_copy(edge_hbm.at[0, pl.ds(base + b * BLK, BLK)],
                            sidx.at[b])
            pltpu.sync_copy(edge_hbm.at[1, pl.ds(base + b * BLK, BLK)],
                            didx.at[b])
        plsc.subcore_barrier()

        def srow(c):
            return sidx.at[(c // BLK) % 2, c % BLK]

        def drow(c):
            return didx.at[(c // BLK) % 2, c % BLK]

        def body(j, carry):
            r = j % RING

            @pl.when((j % BLK == 5) & (j >= BLK + 5)
                     & (j <= (NBLK - 2) * BLK + 5))
            def _():
                b = j // BLK + 1
                slot = b % 2
                pltpu.sync_copy(edge_hbm.at[0, pl.ds(base + b * BLK, BLK)],
                                sidx.at[slot])
                pltpu.sync_copy(edge_hbm.at[1, pl.ds(base + b * BLK, BLK)],
                                didx.at[slot])

            @pl.when(j >= RING)
            def _():
                pltpu.make_async_copy(
                    rows.at[r], acc.at[drow(j - RING)], ssem.at[r]).wait()

            @pl.when(j < NCH_W)
            def _():
                pltpu.async_copy(table_hbm.at[srow(j)], rows.at[r],
                                 gsem.at[r])

            @pl.when(j >= SKEW)
            def _():
                c = j - SKEW
                rc = c % RING
                pltpu.make_async_copy(table_hbm.at[srow(c)], rows.at[rc],
                                      gsem.at[rc]).wait()
                pltpu.async_copy(rows.at[rc], acc.at[drow(c)],
                                 ssem.at[rc], add=True)

            return carry

        lax.fori_loop(0, NCH_W + SKEW, body, 0)
        for t in range(RING - SKEW):
            c = NCH_W - (RING - SKEW) + t
            rc = c % RING
            pltpu.make_async_copy(rows.at[rc], acc.at[drow(c)],
                                  ssem.at[rc]).wait()
        plsc.subcore_barrier()
        pltpu.sync_copy(acc.at[pl.ds(r0, ROWS_PER_TILE)],
                        out_hbm.at[cid, pl.ds(r0, ROWS_PER_TILE)])

    return k


_sc_scatter_1 = _make_sc_scatter(D1, CHUNK1, BLK1)
_sc_scatter_2 = _make_sc_scatter(D2, CHUNK2, BLK2)


def _mm1_body(x_ref, w_ref, o_ref):
    y = jnp.dot(x_ref[...], w_ref[...], preferred_element_type=jnp.float32)
    ones = jnp.ones((y.shape[0], 1), jnp.float32)
    zeros = jnp.zeros((y.shape[0], D1 - 101), jnp.float32)
    o_ref[...] = jnp.concatenate([y, ones, zeros], axis=1)


def _mid_body(acc_ref, y_ref, b1_ref, w2_ref, o_ref):
    s = acc_ref[0] + acc_ref[1]
    deg = s[:, 100:101]
    agg = jnp.where(deg > 0.0,
                    s[:, :100] / jnp.maximum(deg, 1.0),
                    y_ref[:, :100])
    h1 = jnp.maximum(agg + b1_ref[...], 0.0)
    y2 = jnp.dot(h1, w2_ref[...], preferred_element_type=jnp.float32)
    ones = jnp.ones((y2.shape[0], 1), jnp.float32)
    zeros = jnp.zeros((y2.shape[0], D2 - 21), jnp.float32)
    o_ref[...] = jnp.concatenate([y2, ones, zeros], axis=1)


def _fin_body(acc_ref, y_ref, b2_ref, sf_ref, wf1_ref, bf1_ref, wf2_ref,
              bf2_ref, o_ref, scr):
    i = pl.program_id(0)

    @pl.when(i == 0)
    def _():
        scr[...] = jnp.zeros_like(scr)

    s = acc_ref[0] + acc_ref[1]
    deg = s[:, 20:21]
    agg = jnp.where(deg > 0.0,
                    s[:, :20] / jnp.maximum(deg, 1.0),
                    y_ref[:, :20])
    h2 = jnp.maximum(agg + b2_ref[...], 0.0)
    scr[...] += jnp.sum(h2, axis=0, keepdims=True)

    @pl.when(i == pl.num_programs(0) - 1)
    def _():
        hg = scr[...] / jnp.float32(N)
        z = (jnp.dot(hg, wf1_ref[0:20, :], preferred_element_type=jnp.float32)
             + jnp.dot(sf_ref[...], wf1_ref[20:36, :],
                       preferred_element_type=jnp.float32)
             + bf1_ref[...])
        z = jnp.maximum(z, 0.0)
        o_ref[...] = (jnp.dot(z, wf2_ref[...],
                              preferred_element_type=jnp.float32)
                      + bf2_ref[...])


def kernel(x, edge_index, self_feat, W1, b1, W2, b2, Wf1, bf1, Wf2, bf2):
    edges1 = edge_index.reshape(2, E // CHUNK1, CHUNK1)
    edges2 = edge_index.reshape(2, E // CHUNK2, CHUNK2)

    y1p = pl.pallas_call(
        _mm1_body,
        grid=(N // BN,),
        in_specs=[
            pl.BlockSpec((BN, 128), lambda i: (i, 0)),
            pl.BlockSpec((128, 100), lambda i: (0, 0)),
        ],
        out_specs=pl.BlockSpec((BN, D1), lambda i: (i, 0)),
        out_shape=jax.ShapeDtypeStruct((NP, D1), jnp.float32),
    )(x, W1)

    acc1 = _sc_scatter_1(y1p, edges1)

    y2p = pl.pallas_call(
        _mid_body,
        grid=(N // BN,),
        in_specs=[
            pl.BlockSpec((NC, BN, D1), lambda i: (0, i, 0)),
            pl.BlockSpec((BN, D1), lambda i: (i, 0)),
            pl.BlockSpec((1, 100), lambda i: (0, 0)),
            pl.BlockSpec((100, 20), lambda i: (0, 0)),
        ],
        out_specs=pl.BlockSpec((BN, D2), lambda i: (i, 0)),
        out_shape=jax.ShapeDtypeStruct((NP, D2), jnp.float32),
    )(acc1, y1p, b1.reshape(1, 100), W2)

    acc2 = _sc_scatter_2(y2p, edges2)

    out = pl.pallas_call(
        _fin_body,
        grid=(N // BN,),
        in_specs=[
            pl.BlockSpec((NC, BN, D2), lambda i: (0, i, 0)),
            pl.BlockSpec((BN, D2), lambda i: (i, 0)),
            pl.BlockSpec((1, 20), lambda i: (0, 0)),
            pl.BlockSpec((1, 16), lambda i: (0, 0)),
            pl.BlockSpec((36, 10), lambda i: (0, 0)),
            pl.BlockSpec((1, 10), lambda i: (0, 0)),
            pl.BlockSpec((10, 10), lambda i: (0, 0)),
            pl.BlockSpec((1, 10), lambda i: (0, 0)),
        ],
        out_specs=pl.BlockSpec((1, 10), lambda i: (0, 0)),
        out_shape=jax.ShapeDtypeStruct((1, 10), jnp.float32),
        scratch_shapes=[pltpu.VMEM((1, 20), jnp.float32)],
    )(acc2, y2p, b2.reshape(1, 20), self_feat, Wf1, bf1.reshape(1, 10),
      Wf2, bf2.reshape(1, 10))

    return out


# speedup vs baseline: 19.9886x; 1.0146x over previous
import functools

import jax
import jax.numpy as jnp
from jax import lax
from jax.experimental import pallas as pl
from jax.experimental.pallas import tpu as pltpu
from jax.experimental.pallas import tpu_sc as plsc

N = 10000
E = 320000
D1 = 128
D2 = 32
NC = 2
NS = 16
NW = NC * NS
NP = 10112
ROWS_PER_TILE = NP // NS
BN = 2000
CHUNK1, BLK1 = 80, 25
CHUNK2, BLK2 = 125, 16


def _make_sc_scatter(d, CHUNK, BLK, RING, SKEW):
    mesh = plsc.VectorSubcoreMesh(core_axis_name="c", subcore_axis_name="s")
    NCH_W = E // (CHUNK * NW)
    NBLK = NCH_W // BLK
    nzero = CHUNK * d // 16
    lanes_per_row = d // 16
    n_full, rem = divmod(ROWS_PER_TILE, CHUNK)

    @functools.partial(
        pl.kernel,
        mesh=mesh,
        out_type=jax.ShapeDtypeStruct((NC, NP, d), jnp.float32),
        scratch_types=[
            pltpu.VMEM((2, BLK, CHUNK), jnp.int32),
            pltpu.VMEM((2, BLK, CHUNK), jnp.int32),
            pltpu.VMEM((RING, CHUNK, d), jnp.float32),
            pltpu.VMEM_SHARED((NP, d), jnp.float32),
            pltpu.SemaphoreType.DMA((RING,)),
            pltpu.SemaphoreType.DMA((RING,)),
        ],
        compiler_params=pltpu.CompilerParams(use_tc_tiling_on_sc=False),
    )
    def k(table_hbm, edge_hbm, out_hbm, sidx, didx, rows, acc, gsem, ssem):
        cid = lax.axis_index("c")
        sid = lax.axis_index("s")
        wid = sid * NC + cid
        r0 = sid * ROWS_PER_TILE
        base = wid * NCH_W

        def zbody(i, carry):
            rr = i // lanes_per_row
            cc = (i % lanes_per_row) * 16
            rows[0, rr, pl.ds(cc, 16)] = jnp.zeros((16,), jnp.float32)
            return carry

        lax.fori_loop(0, nzero, zbody, 0)
        for kk in range(n_full):
            pltpu.sync_copy(rows.at[0],
                            acc.at[pl.ds(r0 + kk * CHUNK, CHUNK)])
        if rem:
            pltpu.sync_copy(rows.at[0, pl.ds(0, rem)],
                            acc.at[pl.ds(r0 + n_full * CHUNK, rem)])

        for b in range(2):
            pltpu.sync_copy(edge_hbm.at[0, pl.ds(base + b * BLK, BLK)],
                            sidx.at[b])
            pltpu.sync_copy(edge_hbm.at[1, pl.ds(base + b * BLK, BLK)],
                            didx.at[b])
        plsc.subcore_barrier()

        def srow(c):
            return sidx.at[(c // BLK) % 2, c % BLK]

        def drow(c):
            return didx.at[(c // BLK) % 2, c % BLK]

        def body(j, carry):
            r = j % RING

            @pl.when((j % BLK == 5) & (j >= BLK + 5)
                     & (j <= (NBLK - 2) * BLK + 5))
            def _():
                b = j // BLK + 1
                slot = b % 2
                pltpu.sync_copy(edge_hbm.at[0, pl.ds(base + b * BLK, BLK)],
                                sidx.at[slot])
                pltpu.sync_copy(edge_hbm.at[1, pl.ds(base + b * BLK, BLK)],
                                didx.at[slot])

            @pl.when(j >= RING)
            def _():
                pltpu.make_async_copy(
                    rows.at[r], acc.at[drow(j - RING)], ssem.at[r]).wait()

            @pl.when(j < NCH_W)
            def _():
                pltpu.async_copy(table_hbm.at[srow(j)], rows.at[r],
                                 gsem.at[r])

            @pl.when(j >= SKEW)
            def _():
                c = j - SKEW
                rc = c % RING
                pltpu.make_async_copy(table_hbm.at[srow(c)], rows.at[rc],
                                      gsem.at[rc]).wait()
                pltpu.async_copy(rows.at[rc], acc.at[drow(c)],
                                 ssem.at[rc], add=True)

            return carry

        lax.fori_loop(0, NCH_W + SKEW, body, 0)
        for t in range(RING - SKEW):
            c = NCH_W - (RING - SKEW) + t
            rc = c % RING
            pltpu.make_async_copy(rows.at[rc], acc.at[drow(c)],
                                  ssem.at[rc]).wait()
        plsc.subcore_barrier()
        pltpu.sync_copy(acc.at[pl.ds(r0, ROWS_PER_TILE)],
                        out_hbm.at[cid, pl.ds(r0, ROWS_PER_TILE)])

    return k


_sc_scatter_1 = _make_sc_scatter(D1, CHUNK1, BLK1, 4, 3)
_sc_scatter_2 = _make_sc_scatter(D2, CHUNK2, BLK2, 6, 5)


def _mm1_body(x_ref, w_ref, o_ref):
    y = jnp.dot(x_ref[...], w_ref[...], preferred_element_type=jnp.float32)
    ones = jnp.ones((y.shape[0], 1), jnp.float32)
    zeros = jnp.zeros((y.shape[0], D1 - 101), jnp.float32)
    o_ref[...] = jnp.concatenate([y, ones, zeros], axis=1)


def _mid_body(acc_ref, y_ref, b1_ref, w2_ref, o_ref):
    s = acc_ref[0] + acc_ref[1]
    deg = s[:, 100:101]
    agg = jnp.where(deg > 0.0,
                    s[:, :100] / jnp.maximum(deg, 1.0),
                    y_ref[:, :100])
    h1 = jnp.maximum(agg + b1_ref[...], 0.0)
    y2 = jnp.dot(h1, w2_ref[...], preferred_element_type=jnp.float32)
    ones = jnp.ones((y2.shape[0], 1), jnp.float32)
    zeros = jnp.zeros((y2.shape[0], D2 - 21), jnp.float32)
    o_ref[...] = jnp.concatenate([y2, ones, zeros], axis=1)


def _fin_body(acc_ref, y_ref, b2_ref, sf_ref, wf1_ref, bf1_ref, wf2_ref,
              bf2_ref, o_ref, scr):
    i = pl.program_id(0)

    @pl.when(i == 0)
    def _():
        scr[...] = jnp.zeros_like(scr)

    s = acc_ref[0] + acc_ref[1]
    deg = s[:, 20:21]
    agg = jnp.where(deg > 0.0,
                    s[:, :20] / jnp.maximum(deg, 1.0),
                    y_ref[:, :20])
    h2 = jnp.maximum(agg + b2_ref[...], 0.0)
    scr[...] += jnp.sum(h2, axis=0, keepdims=True)

    @pl.when(i == pl.num_programs(0) - 1)
    def _():
        hg = scr[...] / jnp.float32(N)
        z = (jnp.dot(hg, wf1_ref[0:20, :], preferred_element_type=jnp.float32)
             + jnp.dot(sf_ref[...], wf1_ref[20:36, :],
                       preferred_element_type=jnp.float32)
             + bf1_ref[...])
        z = jnp.maximum(z, 0.0)
        o_ref[...] = (jnp.dot(z, wf2_ref[...],
                              preferred_element_type=jnp.float32)
                      + bf2_ref[...])


def kernel(x, edge_index, self_feat, W1, b1, W2, b2, Wf1, bf1, Wf2, bf2):
    edges1 = edge_index.reshape(2, E // CHUNK1, CHUNK1)
    edges2 = edge_index.reshape(2, E // CHUNK2, CHUNK2)

    y1p = pl.pallas_call(
        _mm1_body,
        grid=(N // BN,),
        in_specs=[
            pl.BlockSpec((BN, 128), lambda i: (i, 0)),
            pl.BlockSpec((128, 100), lambda i: (0, 0)),
        ],
        out_specs=pl.BlockSpec((BN, D1), lambda i: (i, 0)),
        out_shape=jax.ShapeDtypeStruct((NP, D1), jnp.float32),
    )(x, W1)

    acc1 = _sc_scatter_1(y1p, edges1)

    y2p = pl.pallas_call(
        _mid_body,
        grid=(N // BN,),
        in_specs=[
            pl.BlockSpec((NC, BN, D1), lambda i: (0, i, 0)),
            pl.BlockSpec((BN, D1), lambda i: (i, 0)),
            pl.BlockSpec((1, 100), lambda i: (0, 0)),
            pl.BlockSpec((100, 20), lambda i: (0, 0)),
        ],
        out_specs=pl.BlockSpec((BN, D2), lambda i: (i, 0)),
        out_shape=jax.ShapeDtypeStruct((NP, D2), jnp.float32),
    )(acc1, y1p, b1.reshape(1, 100), W2)

    acc2 = _sc_scatter_2(y2p, edges2)

    out = pl.pallas_call(
        _fin_body,
        grid=(N // BN,),
        in_specs=[
            pl.BlockSpec((NC, BN, D2), lambda i: (0, i, 0)),
            pl.BlockSpec((BN, D2), lambda i: (i, 0)),
            pl.BlockSpec((1, 20), lambda i: (0, 0)),
            pl.BlockSpec((1, 16), lambda i: (0, 0)),
            pl.BlockSpec((36, 10), lambda i: (0, 0)),
            pl.BlockSpec((1, 10), lambda i: (0, 0)),
            pl.BlockSpec((10, 10), lambda i: (0, 0)),
            pl.BlockSpec((1, 10), lambda i: (0, 0)),
        ],
        out_specs=pl.BlockSpec((1, 10), lambda i: (0, 0)),
        out_shape=jax.ShapeDtypeStruct((1, 10), jnp.float32),
        scratch_shapes=[pltpu.VMEM((1, 20), jnp.float32)],
    )(acc2, y2p, b2.reshape(1, 20), self_feat, Wf1, bf1.reshape(1, 10),
      Wf2, bf2.reshape(1, 10))

    return out
